# Initial kernel scaffold; baseline (speedup 1.0000x reference)
import jax
import jax.numpy as jnp
from jax.experimental import pallas as pl


def kernel(node_s, node_vec, edge, edge_difference, edge_dis, W1, b1, W2, b2, Wf, bf):
    raise NotImplementedError("write your pallas kernel here")


# speedup vs baseline: 1.4154x; 1.4154x over previous
import functools

import jax
import jax.numpy as jnp
from jax import lax
from jax.experimental import pallas as pl
from jax.experimental.pallas import tpu as pltpu
from jax.experimental.pallas import tpu_sc as plsc

F = 128
EDGE_SIZE = 16
CUTOFF = 5.0
N_NODES = 10000
N_EDGES = 320000

NB = 1000
EB = 2000

NSUB = 16
EPW = N_EDGES // NSUB
CHUNK = 80
NITER = EPW // CHUNK
ROWS_A = 624
ROWS_LAST = N_NODES - (NSUB - 1) * ROWS_A
OFF_LAST = (NSUB - 1) * ROWS_A


def _node_tc_kernel(ns_ref, nv_ref, w1_ref, b1_ref, w2_ref, b2_ref,
                    d_ref, base_ref):
    x = ns_ref[...]
    h = jnp.dot(x, w1_ref[...], preferred_element_type=jnp.float32) + b1_ref[...]
    h = h * jax.nn.sigmoid(h)
    s = jnp.dot(h, w2_ref[...], preferred_element_type=jnp.float32) + b2_ref[...]
    nv = nv_ref[...]
    d_ref[0] = s[:, :F] * nv
    d_ref[1] = s[:, F:]
    base_ref[0] = nv
    base_ref[1] = ns_ref[...]


def _edge_tc_kernel(dis_ref, wf_ref, bf_ref, fw_ref):
    d = dis_ref[...]
    n = lax.broadcasted_iota(jnp.int32, (1, EDGE_SIZE), 1).astype(jnp.float32) + 1.0
    rb = jnp.sin(n * (jnp.pi / CUTOFF) * d) / d
    fw = jnp.dot(rb, wf_ref[...], preferred_element_type=jnp.float32) + bf_ref[...]
    cc = jnp.where(d < CUTOFF, 0.5 * jnp.cos(d / CUTOFF) + 1.0, 0.0)
    fw = fw * cc
    fw_ref[0] = fw[:, :F]
    fw_ref[1] = fw[:, F:]


def _sc_body(d2_hbm, fw2_hbm, src_hbm, dst_hbm, base2_hbm, out_hbm,
             dsti_v, srci_v, rows_v, fw_v, acc_sh, sem):
    c = lax.axis_index("c")
    s = lax.axis_index("s")
    cbase = pl.multiple_of(c * N_NODES, 8)

    @pl.when(s < NSUB - 1)
    def _():
        off = pl.multiple_of(s * ROWS_A, 8)
        pltpu.sync_copy(base2_hbm.at[pl.ds(cbase + off, ROWS_A)],
                        acc_sh.at[pl.ds(off, ROWS_A)])

    @pl.when(s == NSUB - 1)
    def _():
        pltpu.sync_copy(base2_hbm.at[pl.ds(cbase + OFF_LAST, ROWS_LAST)],
                        acc_sh.at[pl.ds(OFF_LAST, ROWS_LAST)])

    plsc.subcore_barrier()

    ebase = s * EPW

    def body(i, carry):
        off = pl.multiple_of(ebase + i * CHUNK, 8)
        pltpu.sync_copy(dst_hbm.at[pl.ds(off, CHUNK)], dsti_v)
        pltpu.sync_copy(src_hbm.at[pl.ds(off, CHUNK)], srci_v)
        for j in range(CHUNK // 16):
            sl = pl.ds(j * 16, 16)
            dsti_v[sl] = dsti_v[sl] + cbase
        gat = pltpu.async_copy(d2_hbm.at[dsti_v], rows_v, sem)
        pltpu.sync_copy(
            fw2_hbm.at[pl.ds(pl.multiple_of(c * N_EDGES + off, 8), CHUNK)],
            fw_v)
        gat.wait()

        def mul_row(r, carry2):
            for j in range(F // 16):
                sl = pl.ds(j * 16, 16)
                rows_v[r, sl] = rows_v[r, sl] * fw_v[r, sl]
            return carry2

        lax.fori_loop(0, CHUNK, mul_row, 0, unroll=2)
        pltpu.sync_copy(rows_v, acc_sh.at[srci_v], add=True)
        return carry

    lax.fori_loop(0, NITER, body, 0)
    plsc.subcore_barrier()

    @pl.when(s < NSUB - 1)
    def _():
        off = pl.multiple_of(s * ROWS_A, 8)
        pltpu.sync_copy(acc_sh.at[pl.ds(off, ROWS_A)],
                        out_hbm.at[pl.ds(cbase + off, ROWS_A)])

    @pl.when(s == NSUB - 1)
    def _():
        pltpu.sync_copy(acc_sh.at[pl.ds(OFF_LAST, ROWS_LAST)],
                        out_hbm.at[pl.ds(cbase + OFF_LAST, ROWS_LAST)])


@functools.cache
def _make_sc_kernel():
    mesh = plsc.VectorSubcoreMesh(core_axis_name="c", subcore_axis_name="s")
    return pl.kernel(
        _sc_body,
        out_type=jax.ShapeDtypeStruct((2 * N_NODES, F), jnp.float32),
        mesh=mesh,
        scratch_types=[
            pltpu.VMEM((CHUNK,), jnp.int32),
            pltpu.VMEM((CHUNK,), jnp.int32),
            pltpu.VMEM((CHUNK, F), jnp.float32),
            pltpu.VMEM((CHUNK, F), jnp.float32),
            pltpu.VMEM_SHARED((N_NODES, F), jnp.float32),
            pltpu.SemaphoreType.DMA,
        ],
    )


def kernel(node_s, node_vec, edge, edge_difference, edge_dis, W1, b1, W2, b2,
           Wf, bf):
    w2_sel = jnp.concatenate([W2[:, :F], W2[:, 2 * F:]], axis=1)
    b2_sel = jnp.concatenate([b2[:F], b2[2 * F:]]).reshape(1, 2 * F)
    wf_sel = jnp.concatenate([Wf[:, :F], Wf[:, 2 * F:]], axis=1)
    bf_sel = jnp.concatenate([bf[:F], bf[2 * F:]]).reshape(1, 2 * F)
    b1r = b1.reshape(1, F)

    d2, base2 = pl.pallas_call(
        _node_tc_kernel,
        grid=(N_NODES // NB,),
        in_specs=[
            pl.BlockSpec((NB, F), lambda i: (i, 0)),
            pl.BlockSpec((NB, F), lambda i: (i, 0)),
            pl.BlockSpec((F, F), lambda i: (0, 0)),
            pl.BlockSpec((1, F), lambda i: (0, 0)),
            pl.BlockSpec((F, 2 * F), lambda i: (0, 0)),
            pl.BlockSpec((1, 2 * F), lambda i: (0, 0)),
        ],
        out_specs=[
            pl.BlockSpec((2, NB, F), lambda i: (0, i, 0)),
            pl.BlockSpec((2, NB, F), lambda i: (0, i, 0)),
        ],
        out_shape=[
            jax.ShapeDtypeStruct((2, N_NODES, F), jnp.float32),
            jax.ShapeDtypeStruct((2, N_NODES, F), jnp.float32),
        ],
    )(node_s, node_vec, W1, b1r, w2_sel, b2_sel)

    fw2 = pl.pallas_call(
        _edge_tc_kernel,
        grid=(N_EDGES // EB,),
        in_specs=[
            pl.BlockSpec((EB, 1), lambda i: (i, 0)),
            pl.BlockSpec((EDGE_SIZE, 2 * F), lambda i: (0, 0)),
            pl.BlockSpec((1, 2 * F), lambda i: (0, 0)),
        ],
        out_specs=pl.BlockSpec((2, EB, F), lambda i: (0, i, 0)),
        out_shape=jax.ShapeDtypeStruct((2, N_EDGES, F), jnp.float32),
    )(edge_dis.reshape(N_EDGES, 1), wf_sel, bf_sel)

    src = edge[:, 0]
    dst = edge[:, 1]
    out2 = _make_sc_kernel()(d2.reshape(2 * N_NODES, F),
                             fw2.reshape(2 * N_EDGES, F),
                             src, dst, base2.reshape(2 * N_NODES, F))
    return (out2[:N_NODES], out2[N_NODES:])


# speedup vs baseline: 1.7380x; 1.2280x over previous
import functools

import jax
import jax.numpy as jnp
from jax import lax
from jax.experimental import pallas as pl
from jax.experimental.pallas import tpu as pltpu
from jax.experimental.pallas import tpu_sc as plsc

F = 128
EDGE_SIZE = 16
CUTOFF = 5.0
N_NODES = 10000
N_EDGES = 320000

NB = 1000
EB = 2000

NSUB = 16
EPW = N_EDGES // NSUB
CHUNK = 80
NITER = EPW // CHUNK
GROUP = 10
NGROUP = NITER // GROUP
ROWS_A = 624
ROWS_LAST = N_NODES - (NSUB - 1) * ROWS_A
OFF_LAST = (NSUB - 1) * ROWS_A


def _node_tc_kernel(ns_ref, nv_ref, w1_ref, b1_ref, w2_ref, b2_ref,
                    d_ref, base_ref):
    x = ns_ref[...]
    h = jnp.dot(x, w1_ref[...], preferred_element_type=jnp.float32) + b1_ref[...]
    h = h * jax.nn.sigmoid(h)
    s = jnp.dot(h, w2_ref[...], preferred_element_type=jnp.float32) + b2_ref[...]
    nv = nv_ref[...]
    d_ref[0] = s[:, :F] * nv
    d_ref[1] = s[:, F:]
    base_ref[0] = nv
    base_ref[1] = ns_ref[...]


def _edge_tc_kernel(dis_ref, wf_ref, bf_ref, fw_ref):
    d = dis_ref[...]
    n = lax.broadcasted_iota(jnp.int32, (1, EDGE_SIZE), 1).astype(jnp.float32) + 1.0
    rb = jnp.sin(n * (jnp.pi / CUTOFF) * d) / d
    fw = jnp.dot(rb, wf_ref[...], preferred_element_type=jnp.float32) + bf_ref[...]
    cc = jnp.where(d < CUTOFF, 0.5 * jnp.cos(d / CUTOFF) + 1.0, 0.0)
    fw = fw * cc
    fw_ref[0] = fw[:, :F]
    fw_ref[1] = fw[:, F:]


def _sc_body(d2_hbm, fw2_hbm, src3_hbm, dst3_hbm, base2_hbm, out_hbm,
             srcs_v, dsts_v, rows_v, fw_v, acc_sh, gsem, fsem, ssem):
    c = lax.axis_index("c")
    s = lax.axis_index("s")
    cbase = pl.multiple_of(c * N_NODES, 8)
    ebase = s * EPW

    def load_group(g, slot):
        pltpu.sync_copy(src3_hbm.at[s, g], srcs_v.at[slot])
        pltpu.sync_copy(dst3_hbm.at[s, g], dsts_v.at[slot])

        def bias_row(r, carry):
            for j in range(CHUNK // 16):
                sl = pl.ds(j * 16, 16)
                dsts_v[slot, r, sl] = dsts_v[slot, r, sl] + cbase
            return carry

        lax.fori_loop(0, GROUP, bias_row, 0, unroll=2)

    @pl.when(s < NSUB - 1)
    def _():
        off = pl.multiple_of(s * ROWS_A, 8)
        pltpu.sync_copy(base2_hbm.at[pl.ds(cbase + off, ROWS_A)],
                        acc_sh.at[pl.ds(off, ROWS_A)])

    @pl.when(s == NSUB - 1)
    def _():
        pltpu.sync_copy(base2_hbm.at[pl.ds(cbase + OFF_LAST, ROWS_LAST)],
                        acc_sh.at[pl.ds(OFF_LAST, ROWS_LAST)])

    plsc.subcore_barrier()

    def islot(i):
        return lax.rem(lax.div(i, GROUP), 2)

    def irow(i):
        return lax.rem(i, GROUP)

    def start_loads(i, b):
        pltpu.async_copy(d2_hbm.at[dsts_v.at[islot(i), irow(i)]],
                         rows_v.at[b], gsem.at[b])
        off = pl.multiple_of(c * N_EDGES + ebase, 8) + i * CHUNK
        pltpu.async_copy(fw2_hbm.at[pl.ds(off, CHUNK)], fw_v.at[b], fsem.at[b])

    def wait_loads(i, b):
        pltpu.make_async_copy(d2_hbm.at[dsts_v.at[islot(i), irow(i)]],
                              rows_v.at[b], gsem.at[b]).wait()
        off = pl.multiple_of(c * N_EDGES + ebase, 8) + i * CHUNK
        pltpu.make_async_copy(fw2_hbm.at[pl.ds(off, CHUNK)], fw_v.at[b],
                              fsem.at[b]).wait()

    def start_scatter(i, b):
        pltpu.async_copy(rows_v.at[b], acc_sh.at[srcs_v.at[islot(i), irow(i)]],
                         ssem.at[b], add=True)

    def wait_scatter(i, b):
        pltpu.make_async_copy(rows_v.at[b],
                              acc_sh.at[srcs_v.at[islot(i), irow(i)]],
                              ssem.at[b]).wait()

    load_group(0, 0)
    start_loads(0, 0)

    def body(t, carry):
        for b in range(2):
            i = 2 * t + b
            @pl.when(i >= 1)
            def _():
                wait_scatter(i - 1, 1 - b)

            @pl.when(i + 1 < NITER)
            def _():
                @pl.when(lax.rem(i + 1, GROUP) == 0)
                def _():
                    g = lax.div(i + 1, GROUP)
                    load_group(g, lax.rem(g, 2))

                start_loads(i + 1, 1 - b)

            wait_loads(i, b)

            def mul_row(r, carry2):
                for j in range(F // 16):
                    sl = pl.ds(j * 16, 16)
                    rows_v[b, r, sl] = rows_v[b, r, sl] * fw_v[b, r, sl]
                return carry2

            lax.fori_loop(0, CHUNK, mul_row, 0, unroll=2)
            start_scatter(i, b)
        return carry

    lax.fori_loop(0, NITER // 2, body, 0)
    wait_scatter(NITER - 1, 1)

    plsc.subcore_barrier()

    @pl.when(s < NSUB - 1)
    def _():
        off = pl.multiple_of(s * ROWS_A, 8)
        pltpu.sync_copy(acc_sh.at[pl.ds(off, ROWS_A)],
                        out_hbm.at[pl.ds(cbase + off, ROWS_A)])

    @pl.when(s == NSUB - 1)
    def _():
        pltpu.sync_copy(acc_sh.at[pl.ds(OFF_LAST, ROWS_LAST)],
                        out_hbm.at[pl.ds(cbase + OFF_LAST, ROWS_LAST)])


@functools.cache
def _make_sc_kernel():
    mesh = plsc.VectorSubcoreMesh(core_axis_name="c", subcore_axis_name="s")
    return pl.kernel(
        _sc_body,
        out_type=jax.ShapeDtypeStruct((2 * N_NODES, F), jnp.float32),
        mesh=mesh,
        scratch_types=[
            pltpu.VMEM((2, GROUP, CHUNK), jnp.int32),
            pltpu.VMEM((2, GROUP, CHUNK), jnp.int32),
            pltpu.VMEM((2, CHUNK, F), jnp.float32),
            pltpu.VMEM((2, CHUNK, F), jnp.float32),
            pltpu.VMEM_SHARED((N_NODES, F), jnp.float32),
            pltpu.SemaphoreType.DMA((2,)),
            pltpu.SemaphoreType.DMA((2,)),
            pltpu.SemaphoreType.DMA((2,)),
        ],
    )


def kernel(node_s, node_vec, edge, edge_difference, edge_dis, W1, b1, W2, b2,
           Wf, bf):
    w2_sel = jnp.concatenate([W2[:, :F], W2[:, 2 * F:]], axis=1)
    b2_sel = jnp.concatenate([b2[:F], b2[2 * F:]]).reshape(1, 2 * F)
    wf_sel = jnp.concatenate([Wf[:, :F], Wf[:, 2 * F:]], axis=1)
    bf_sel = jnp.concatenate([bf[:F], bf[2 * F:]]).reshape(1, 2 * F)
    b1r = b1.reshape(1, F)

    d2, base2 = pl.pallas_call(
        _node_tc_kernel,
        grid=(N_NODES // NB,),
        in_specs=[
            pl.BlockSpec((NB, F), lambda i: (i, 0)),
            pl.BlockSpec((NB, F), lambda i: (i, 0)),
            pl.BlockSpec((F, F), lambda i: (0, 0)),
            pl.BlockSpec((1, F), lambda i: (0, 0)),
            pl.BlockSpec((F, 2 * F), lambda i: (0, 0)),
            pl.BlockSpec((1, 2 * F), lambda i: (0, 0)),
        ],
        out_specs=[
            pl.BlockSpec((2, NB, F), lambda i: (0, i, 0)),
            pl.BlockSpec((2, NB, F), lambda i: (0, i, 0)),
        ],
        out_shape=[
            jax.ShapeDtypeStruct((2, N_NODES, F), jnp.float32),
            jax.ShapeDtypeStruct((2, N_NODES, F), jnp.float32),
        ],
    )(node_s, node_vec, W1, b1r, w2_sel, b2_sel)

    fw2 = pl.pallas_call(
        _edge_tc_kernel,
        grid=(N_EDGES // EB,),
        in_specs=[
            pl.BlockSpec((EB, 1), lambda i: (i, 0)),
            pl.BlockSpec((EDGE_SIZE, 2 * F), lambda i: (0, 0)),
            pl.BlockSpec((1, 2 * F), lambda i: (0, 0)),
        ],
        out_specs=pl.BlockSpec((2, EB, F), lambda i: (0, i, 0)),
        out_shape=jax.ShapeDtypeStruct((2, N_EDGES, F), jnp.float32),
    )(edge_dis.reshape(N_EDGES, 1), wf_sel, bf_sel)

    src3 = edge[:, 0].reshape(NSUB, NGROUP, GROUP, CHUNK)
    dst3 = edge[:, 1].reshape(NSUB, NGROUP, GROUP, CHUNK)
    out2 = _make_sc_kernel()(d2.reshape(2 * N_NODES, F),
                             fw2.reshape(2 * N_EDGES, F),
                             src3, dst3, base2.reshape(2 * N_NODES, F))
    return (out2[:N_NODES], out2[N_NODES:])


# speedup vs baseline: 1.7429x; 1.0028x over previous
import functools

import jax
import jax.numpy as jnp
from jax import lax
from jax.experimental import pallas as pl
from jax.experimental.pallas import tpu as pltpu
from jax.experimental.pallas import tpu_sc as plsc

F = 128
EDGE_SIZE = 16
CUTOFF = 5.0
N_NODES = 10000
N_EDGES = 320000

NB = 1000
EB = 2000

NSUB = 16
EPW = N_EDGES // NSUB
CHUNK = 40
NITER = EPW // CHUNK
GROUP = 20
NGROUP = NITER // GROUP
ROWS_A = 624
ROWS_LAST = N_NODES - (NSUB - 1) * ROWS_A
OFF_LAST = (NSUB - 1) * ROWS_A


def _node_tc_kernel(ns_ref, nv_ref, w1_ref, b1_ref, w2_ref, b2_ref,
                    d_ref, base_ref):
    x = ns_ref[...]
    h = jnp.dot(x, w1_ref[...], preferred_element_type=jnp.float32) + b1_ref[...]
    h = h * jax.nn.sigmoid(h)
    s = jnp.dot(h, w2_ref[...], preferred_element_type=jnp.float32) + b2_ref[...]
    nv = nv_ref[...]
    d_ref[0] = s[:, :F] * nv
    d_ref[1] = s[:, F:]
    base_ref[0] = nv
    base_ref[1] = ns_ref[...]


def _edge_tc_kernel(dis_ref, wf_ref, bf_ref, fw_ref):
    d = dis_ref[...]
    n = lax.broadcasted_iota(jnp.int32, (1, EDGE_SIZE), 1).astype(jnp.float32) + 1.0
    rb = jnp.sin(n * (jnp.pi / CUTOFF) * d) / d
    fw = jnp.dot(rb, wf_ref[...], preferred_element_type=jnp.float32) + bf_ref[...]
    cc = jnp.where(d < CUTOFF, 0.5 * jnp.cos(d / CUTOFF) + 1.0, 0.0)
    fw = fw * cc
    fw_ref[0] = fw[:, :F]
    fw_ref[1] = fw[:, F:]


def _sc_body(d2_hbm, fw2_hbm, src4_hbm, dst4_hbm, base2_hbm, out2_hbm,
             srcs_v, dsts_v, rows_v, fwb_v, prod_v, acc_sh, gsem, fsem, ssem):
    c = lax.axis_index("c")
    s = lax.axis_index("s")
    cbase = pl.multiple_of(c * N_NODES, 8)
    ebase = s * EPW

    def load_group(g, slot):
        pltpu.sync_copy(src4_hbm.at[s, g], srcs_v.at[slot])
        pltpu.sync_copy(dst4_hbm.at[c, s, g], dsts_v.at[slot])

    @pl.when(s < NSUB - 1)
    def _():
        off = pl.multiple_of(s * ROWS_A, 8)
        pltpu.sync_copy(base2_hbm.at[pl.ds(cbase + off, ROWS_A)],
                        acc_sh.at[pl.ds(off, ROWS_A)])

    @pl.when(s == NSUB - 1)
    def _():
        pltpu.sync_copy(base2_hbm.at[pl.ds(cbase + OFF_LAST, ROWS_LAST)],
                        acc_sh.at[pl.ds(OFF_LAST, ROWS_LAST)])

    plsc.subcore_barrier()

    def islot(i):
        return lax.rem(lax.div(i, GROUP), 2)

    def irow(i):
        return lax.rem(i, GROUP)

    def start_loads(i, b):
        pltpu.async_copy(d2_hbm.at[dsts_v.at[islot(i), irow(i)]],
                         rows_v.at[b], gsem.at[b])
        off = pl.multiple_of(c * N_EDGES + ebase, 8) + i * CHUNK
        pltpu.async_copy(fw2_hbm.at[pl.ds(off, CHUNK)], fwb_v.at[b],
                         fsem.at[b])

    def wait_loads(i, b):
        pltpu.make_async_copy(d2_hbm.at[dsts_v.at[islot(i), irow(i)]],
                              rows_v.at[b], gsem.at[b]).wait()
        off = pl.multiple_of(c * N_EDGES + ebase, 8) + i * CHUNK
        pltpu.make_async_copy(fw2_hbm.at[pl.ds(off, CHUNK)], fwb_v.at[b],
                              fsem.at[b]).wait()

    def start_scatter(i, b):
        pltpu.async_copy(prod_v.at[b], acc_sh.at[srcs_v.at[islot(i), irow(i)]],
                         ssem.at[b], add=True)

    def wait_scatter(i, b):
        pltpu.make_async_copy(prod_v.at[b],
                              acc_sh.at[srcs_v.at[islot(i), irow(i)]],
                              ssem.at[b]).wait()

    load_group(0, 0)
    start_loads(0, 0)

    def body(t, carry):
        for b in range(2):
            i = 2 * t + b
            @pl.when(i + 1 < NITER)
            def _():
                @pl.when(lax.rem(i + 1, GROUP) == 0)
                def _():
                    g = lax.div(i + 1, GROUP)
                    load_group(g, lax.rem(g, 2))

                start_loads(i + 1, 1 - b)

            wait_loads(i, b)

            @pl.when(i >= 2)
            def _():
                wait_scatter(i - 2, b)

            def mul_row(r, carry2):
                for j in range(F // 16):
                    sl = pl.ds(j * 16, 16)
                    prod_v[b, r, sl] = rows_v[b, r, sl] * fwb_v[b, r, sl]
                return carry2

            lax.fori_loop(0, CHUNK, mul_row, 0, unroll=2)
            start_scatter(i, b)
        return carry

    lax.fori_loop(0, NITER // 2, body, 0)
    wait_scatter(NITER - 2, 0)
    wait_scatter(NITER - 1, 1)

    plsc.subcore_barrier()

    @pl.when(s < NSUB - 1)
    def _():
        off = pl.multiple_of(s * ROWS_A, 8)
        pltpu.sync_copy(acc_sh.at[pl.ds(off, ROWS_A)],
                        out2_hbm.at[pl.ds(cbase + off, ROWS_A)])

    @pl.when(s == NSUB - 1)
    def _():
        pltpu.sync_copy(acc_sh.at[pl.ds(OFF_LAST, ROWS_LAST)],
                        out2_hbm.at[pl.ds(cbase + OFF_LAST, ROWS_LAST)])


@functools.cache
def _make_sc_kernel():
    mesh = plsc.VectorSubcoreMesh(core_axis_name="c", subcore_axis_name="s")
    return pl.kernel(
        _sc_body,
        out_type=jax.ShapeDtypeStruct((2 * N_NODES, F), jnp.float32),
        mesh=mesh,
        scratch_types=[
            pltpu.VMEM((2, GROUP, CHUNK), jnp.int32),
            pltpu.VMEM((2, GROUP, CHUNK), jnp.int32),
            pltpu.VMEM((2, CHUNK, F), jnp.float32),
            pltpu.VMEM((2, CHUNK, F), jnp.float32),
            pltpu.VMEM((2, CHUNK, F), jnp.float32),
            pltpu.VMEM_SHARED((N_NODES, F), jnp.float32),
            pltpu.SemaphoreType.DMA((2,)),
            pltpu.SemaphoreType.DMA((2,)),
            pltpu.SemaphoreType.DMA((2,)),
        ],
    )


def kernel(node_s, node_vec, edge, edge_difference, edge_dis, W1, b1, W2, b2,
           Wf, bf):
    w2_sel = jnp.concatenate([W2[:, :F], W2[:, 2 * F:]], axis=1)
    b2_sel = jnp.concatenate([b2[:F], b2[2 * F:]]).reshape(1, 2 * F)
    wf_sel = jnp.concatenate([Wf[:, :F], Wf[:, 2 * F:]], axis=1)
    bf_sel = jnp.concatenate([bf[:F], bf[2 * F:]]).reshape(1, 2 * F)
    b1r = b1.reshape(1, F)

    d2, base2 = pl.pallas_call(
        _node_tc_kernel,
        grid=(N_NODES // NB,),
        in_specs=[
            pl.BlockSpec((NB, F), lambda i: (i, 0)),
            pl.BlockSpec((NB, F), lambda i: (i, 0)),
            pl.BlockSpec((F, F), lambda i: (0, 0)),
            pl.BlockSpec((1, F), lambda i: (0, 0)),
            pl.BlockSpec((F, 2 * F), lambda i: (0, 0)),
            pl.BlockSpec((1, 2 * F), lambda i: (0, 0)),
        ],
        out_specs=[
            pl.BlockSpec((2, NB, F), lambda i: (0, i, 0)),
            pl.BlockSpec((2, NB, F), lambda i: (0, i, 0)),
        ],
        out_shape=[
            jax.ShapeDtypeStruct((2, N_NODES, F), jnp.float32),
            jax.ShapeDtypeStruct((2, N_NODES, F), jnp.float32),
        ],
    )(node_s, node_vec, W1, b1r, w2_sel, b2_sel)

    fw2 = pl.pallas_call(
        _edge_tc_kernel,
        grid=(N_EDGES // EB,),
        in_specs=[
            pl.BlockSpec((EB, 1), lambda i: (i, 0)),
            pl.BlockSpec((EDGE_SIZE, 2 * F), lambda i: (0, 0)),
            pl.BlockSpec((1, 2 * F), lambda i: (0, 0)),
        ],
        out_specs=pl.BlockSpec((2, EB, F), lambda i: (0, i, 0)),
        out_shape=jax.ShapeDtypeStruct((2, N_EDGES, F), jnp.float32),
    )(edge_dis.reshape(N_EDGES, 1), wf_sel, bf_sel)

    src4 = edge[:, 0].reshape(NSUB, NGROUP, GROUP, CHUNK)
    dst = edge[:, 1]
    dst5 = jnp.stack([dst, dst + N_NODES]).reshape(
        2, NSUB, NGROUP, GROUP, CHUNK)
    out2 = _make_sc_kernel()(d2.reshape(2 * N_NODES, F),
                             fw2.reshape(2 * N_EDGES, F),
                             src4, dst5, base2.reshape(2 * N_NODES, F))
    return (out2[:N_NODES], out2[N_NODES:])
